# baseline (speedup 1.0000x reference)
import functools

import jax
import jax.numpy as jnp
from jax import lax
from jax.experimental import pallas as pl
from jax.experimental.pallas import tpu as pltpu
from jax.experimental.pallas import tpu_sc as plsc

_NUM_USERS = 1000000
_DIM = 16
_BATCH = 16384
_NC = 2
_NS = 16
_NW = _NC * _NS
_BPW = _BATCH // _NW
_CHUNK = 128
_NCH = _BPW // _CHUNK

_LC = 2048
_NFULL = _NUM_USERS // _LC
_ALIGNED_END = (_NUM_USERS // 128) * 128
_TAIL = _ALIGNED_END - _NFULL * _LC
_RAG = _NUM_USERS - _ALIGNED_END
_NTC = -(-_NUM_USERS // 128)
_ROWS = _NTC * _DIM

_mesh = plsc.VectorSubcoreMesh(core_axis_name="c", subcore_axis_name="s")


_KB = 8192
_NB = -(-_NUM_USERS // _KB)
_TROWS = _NB * (_KB // 128) * _DIM


def _repack_body(w_ref, o_ref):
    for t0 in range(0, _KB // 128, 8):
        for t in range(t0, t0 + 8):
            o_ref[pl.ds(t * _DIM, _DIM), :] = w_ref[:, pl.ds(t * 128, 128)]


_repack = pl.pallas_call(
    _repack_body,
    grid=(_NB,),
    in_specs=[pl.BlockSpec((_DIM, _KB), lambda c: (0, c))],
    out_specs=pl.BlockSpec((_KB // 128 * _DIM, 128), lambda c: (c, 0)),
    out_shape=jax.ShapeDtypeStruct((_TROWS, 128), jnp.float32),
)


@functools.partial(
    pl.kernel,
    mesh=_mesh,
    out_type=jax.ShapeDtypeStruct((_DIM, _BATCH), jnp.float32),
    scratch_types=[
        pltpu.VMEM((_BPW,), jnp.int32),
        pltpu.VMEM((_DIM, _BPW), jnp.int32),
        pltpu.VMEM((_DIM, _BPW), jnp.float32),
        pltpu.SemaphoreType.DMA,
    ],
)
def _lookup(wlin_hbm, x_hbm, out_hbm, xv, idxv, rowsd, sem):
    wid = lax.axis_index("s") * _NC + lax.axis_index("c")
    base = wid * _BPW
    pltpu.sync_copy(x_hbm.at[pl.ds(base, _BPW)], xv)

    def offsets(c, carry):
        vx = xv[pl.ds(c * _NS, _NS)]
        vt = (vx >> 7) * 2048 + (vx & 127)
        for d in range(_DIM):
            idxv[d, pl.ds(c * _NS, _NS)] = vt + ((d // 8) * 1024 + (d % 8) * 128)
        return carry

    lax.fori_loop(0, _BPW // _NS, offsets, 0)

    copies = [
        pltpu.async_copy(
            wlin_hbm.at[idxv.at[d, pl.ds(k * _CHUNK, _CHUNK)]],
            rowsd.at[d, pl.ds(k * _CHUNK, _CHUNK)],
            sem,
        )
        for d in range(_DIM)
        for k in range(_NCH)
    ]
    for cp in copies:
        cp.wait()

    pltpu.sync_copy(rowsd, out_hbm.at[:, pl.ds(base, _BPW)])


def kernel(x, W):
    wlin = _repack(W).reshape(-1)
    h = _lookup(wlin, x.astype(jnp.int32))
    return h.T


# speedup vs baseline: 1.6151x; 1.6151x over previous
import functools

import jax
import jax.numpy as jnp
from jax import lax
from jax.experimental import pallas as pl
from jax.experimental.pallas import tpu as pltpu
from jax.experimental.pallas import tpu_sc as plsc

_NUM_USERS = 1000000
_DIM = 16
_BATCH = 16384
_NC = 2
_NS = 16
_NW = _NC * _NS
_BPW = _BATCH // _NW
_CHUNK = 128
_NCH = _BPW // _CHUNK

_LC = 2048
_NFULL = _NUM_USERS // _LC
_ALIGNED_END = (_NUM_USERS // 128) * 128
_TAIL = _ALIGNED_END - _NFULL * _LC
_RAG = _NUM_USERS - _ALIGNED_END
_NTC = -(-_NUM_USERS // 128)
_ROWS = _NTC * _DIM

_mesh = plsc.VectorSubcoreMesh(core_axis_name="c", subcore_axis_name="s")


_KB = 32768
_NB = -(-_NUM_USERS // _KB)
_TROWS = _NB * (_KB // 128) * _DIM


def _repack_body(w_ref, o_ref):
    for t0 in range(0, _KB // 128, 8):
        for t in range(t0, t0 + 8):
            o_ref[pl.ds(t * _DIM, _DIM), :] = w_ref[:, pl.ds(t * 128, 128)]


_repack = pl.pallas_call(
    _repack_body,
    grid=(_NB,),
    in_specs=[pl.BlockSpec((_DIM, _KB), lambda c: (0, c))],
    out_specs=pl.BlockSpec((_KB // 128 * _DIM, 128), lambda c: (c, 0)),
    out_shape=jax.ShapeDtypeStruct((_TROWS, 128), jnp.float32),
)


@functools.partial(
    pl.kernel,
    mesh=_mesh,
    out_type=jax.ShapeDtypeStruct((_DIM, _BATCH), jnp.float32),
    scratch_types=[
        pltpu.VMEM((_BPW,), jnp.int32),
        pltpu.VMEM((_DIM, _BPW), jnp.int32),
        pltpu.VMEM((_DIM, _BPW), jnp.float32),
        pltpu.SemaphoreType.DMA,
    ],
)
def _lookup(wlin_hbm, x_hbm, out_hbm, xv, idxv, rowsd, sem):
    wid = lax.axis_index("s") * _NC + lax.axis_index("c")
    base = wid * _BPW
    pltpu.sync_copy(x_hbm.at[pl.ds(base, _BPW)], xv)

    def offsets(c, carry):
        vx = xv[pl.ds(c * _NS, _NS)]
        vt = (vx >> 7) * 2048 + (vx & 127)
        for d in range(_DIM):
            idxv[d, pl.ds(c * _NS, _NS)] = vt + ((d // 8) * 1024 + (d % 8) * 128)
        return carry

    lax.fori_loop(0, _BPW // _NS, offsets, 0)

    copies = [
        pltpu.async_copy(
            wlin_hbm.at[idxv.at[d, pl.ds(k * _CHUNK, _CHUNK)]],
            rowsd.at[d, pl.ds(k * _CHUNK, _CHUNK)],
            sem,
        )
        for d in range(_DIM)
        for k in range(_NCH)
    ]
    for cp in copies:
        cp.wait()

    pltpu.sync_copy(rowsd, out_hbm.at[:, pl.ds(base, _BPW)])


def kernel(x, W):
    wlin = _repack(W).reshape(-1)
    h = _lookup(wlin, x.astype(jnp.int32))
    return h.T


# speedup vs baseline: 1.7480x; 1.0823x over previous
import functools

import jax
import jax.numpy as jnp
from jax import lax
from jax.experimental import pallas as pl
from jax.experimental.pallas import tpu as pltpu
from jax.experimental.pallas import tpu_sc as plsc

_NUM_USERS = 1000000
_DIM = 16
_BATCH = 16384
_NC = 2
_NS = 16
_NW = _NC * _NS
_BPW = _BATCH // _NW
_CHUNK = 128
_NCH = _BPW // _CHUNK

_LC = 2048
_NFULL = _NUM_USERS // _LC
_ALIGNED_END = (_NUM_USERS // 128) * 128
_TAIL = _ALIGNED_END - _NFULL * _LC
_RAG = _NUM_USERS - _ALIGNED_END
_NTC = -(-_NUM_USERS // 128)
_ROWS = _NTC * _DIM

_mesh = plsc.VectorSubcoreMesh(core_axis_name="c", subcore_axis_name="s")


_KB = 131072
_NB = -(-_NUM_USERS // _KB)
_TROWS = _NB * (_KB // 128) * _DIM


def _repack_body(w_ref, o_ref):
    for t0 in range(0, _KB // 128, 8):
        for t in range(t0, t0 + 8):
            o_ref[pl.ds(t * _DIM, _DIM), :] = w_ref[:, pl.ds(t * 128, 128)]


_repack = pl.pallas_call(
    _repack_body,
    grid=(_NB,),
    in_specs=[pl.BlockSpec((_DIM, _KB), lambda c: (0, c))],
    out_specs=pl.BlockSpec((_KB // 128 * _DIM, 128), lambda c: (c, 0)),
    out_shape=jax.ShapeDtypeStruct((_TROWS, 128), jnp.float32),
)


@functools.partial(
    pl.kernel,
    mesh=_mesh,
    out_type=jax.ShapeDtypeStruct((_DIM, _BATCH), jnp.float32),
    scratch_types=[
        pltpu.VMEM((_BPW,), jnp.int32),
        pltpu.VMEM((_DIM, _BPW), jnp.int32),
        pltpu.VMEM((_DIM, _BPW), jnp.float32),
        pltpu.SemaphoreType.DMA,
    ],
)
def _lookup(wlin_hbm, x_hbm, out_hbm, xv, idxv, rowsd, sem):
    wid = lax.axis_index("s") * _NC + lax.axis_index("c")
    base = wid * _BPW
    pltpu.sync_copy(x_hbm.at[pl.ds(base, _BPW)], xv)

    def offsets(c, carry):
        vx = xv[pl.ds(c * _NS, _NS)]
        vt = (vx >> 7) * 2048 + (vx & 127)
        for d in range(_DIM):
            idxv[d, pl.ds(c * _NS, _NS)] = vt + ((d // 8) * 1024 + (d % 8) * 128)
        return carry

    lax.fori_loop(0, _BPW // _NS, offsets, 0)

    copies = [
        pltpu.async_copy(
            wlin_hbm.at[idxv.at[d, pl.ds(k * _CHUNK, _CHUNK)]],
            rowsd.at[d, pl.ds(k * _CHUNK, _CHUNK)],
            sem,
        )
        for d in range(_DIM)
        for k in range(_NCH)
    ]
    for cp in copies:
        cp.wait()

    pltpu.sync_copy(rowsd, out_hbm.at[:, pl.ds(base, _BPW)])


def kernel(x, W):
    wlin = _repack(W).reshape(-1)
    h = _lookup(wlin, x.astype(jnp.int32))
    return h.T
